# baseline (speedup 1.0000x reference)
import functools

import numpy as np
import jax
import jax.numpy as jnp
from jax import lax
from jax.experimental import pallas as pl
from jax.experimental.pallas import tpu as pltpu
from jax.experimental.pallas import tpu_sc as plsc

D_MODEL = 1024
MAX_LEN = 8192
BATCH = 4
SEQ_LEN = 4096
NUM_CORES = 2
NUM_SUBCORES = 16
NW = NUM_CORES * NUM_SUBCORES
S_PER_W = SEQ_LEN // NW
CHUNK = 16
NJ = S_PER_W // CHUNK
NITER = NJ * BATCH
LANES = 16


def _sinusoid_pe_np(max_len, d_model):
    pos = np.arange(max_len, dtype=np.float32)[:, None]
    i = np.arange(0, d_model, 2, dtype=np.float32)
    div = np.power(10000.0, i / d_model)
    pe = np.zeros((max_len, d_model), dtype=np.float32)
    pe[:, 0::2] = np.sin(pos / div)
    pe[:, 1::2] = np.cos(pos / div)
    return pe


_PE = _sinusoid_pe_np(MAX_LEN, D_MODEL)[:SEQ_LEN].astype(np.float32)


@functools.partial(
    pl.kernel,
    out_type=jax.ShapeDtypeStruct((BATCH * SEQ_LEN, D_MODEL), jnp.float32),
    mesh=plsc.VectorSubcoreMesh(core_axis_name="c", subcore_axis_name="s"),
    scratch_types=[
        pltpu.VMEM((BATCH, NJ, CHUNK), jnp.int32),
        pltpu.VMEM((CHUNK, D_MODEL), jnp.float32),
        pltpu.VMEM((CHUNK, D_MODEL), jnp.float32),
        pltpu.VMEM((CHUNK, D_MODEL), jnp.float32),
        pltpu.VMEM((CHUNK, D_MODEL), jnp.float32),
        pltpu.VMEM((CHUNK, D_MODEL), jnp.float32),
        pltpu.SemaphoreType.DMA,
        pltpu.SemaphoreType.DMA,
        pltpu.SemaphoreType.DMA,
        pltpu.SemaphoreType.DMA,
        pltpu.SemaphoreType.DMA,
        pltpu.SemaphoreType.DMA,
        pltpu.SemaphoreType.DMA,
        pltpu.SemaphoreType.DMA,
    ],
)
def _emb_kernel(x_hbm, table_hbm, pe_hbm, out_hbm,
                idx_v, rows0, rows1, rows2, pe0, pe1,
                g0, g1, g2, s0, s1, s2, p0, p1):
    bufs = (rows0, rows1, rows2)
    gsems = (g0, g1, g2)
    ssems = (s0, s1, s2)
    pebs = (pe0, pe1)
    psems = (p0, p1)

    wid = lax.axis_index("s") * NUM_CORES + lax.axis_index("c")
    s_lo = wid * S_PER_W

    pltpu.sync_copy(x_hbm.at[wid], idx_v)

    def gather(n, handles):
        b, j = n % BATCH, n // BATCH
        handles[n % 3] = pltpu.async_copy(
            table_hbm.at[idx_v.at[b, j]], bufs[n % 3], gsems[n % 3])

    def pe_load(j, handles):
        handles[j % 2] = pltpu.async_copy(
            pe_hbm.at[pl.ds(s_lo + j * CHUNK, CHUNK)], pebs[j % 2],
            psems[j % 2])

    ghandles = [None, None, None]
    shandles = [None, None, None]
    phandles = [None, None]
    pe_load(0, phandles)
    gather(0, ghandles)

    for n in range(NITER):
        b, j = n % BATCH, n // BATCH
        buf = bufs[n % 3]
        peb = pebs[j % 2]
        if b == 0:
            phandles[j % 2].wait()
            if j + 1 < NJ:
                pe_load(j + 1, phandles)
        if shandles[(n + 1) % 3] is not None:
            shandles[(n + 1) % 3].wait()
            shandles[(n + 1) % 3] = None
        if n + 1 < NITER:
            gather(n + 1, ghandles)
        ghandles[n % 3].wait()

        def add_row(r, carry):
            def add_grp(g, carry2):
                base = g * (16 * LANES)
                for c in range(16):
                    sl = pl.ds(base + c * LANES, LANES)
                    plsc.addupdate(buf.at[r, sl], peb[r, sl])
                return carry2

            lax.fori_loop(0, D_MODEL // (16 * LANES), add_grp, 0)
            return carry

        lax.fori_loop(0, CHUNK, add_row, 0)
        shandles[n % 3] = pltpu.async_copy(
            buf, out_hbm.at[pl.ds(b * SEQ_LEN + s_lo + j * CHUNK, CHUNK)],
            ssems[n % 3])

    for h in shandles:
        if h is not None:
            h.wait()


def kernel(x, tok_table):
    x_grouped = x.reshape(BATCH, NW, NJ, CHUNK).transpose(1, 0, 2, 3)
    pe = jnp.asarray(_PE)
    out = _emb_kernel(x_grouped, tok_table, pe)
    return out.reshape(BATCH, SEQ_LEN, D_MODEL)
